# Initial kernel scaffold; baseline (speedup 1.0000x reference)
import jax
import jax.numpy as jnp
from jax.experimental import pallas as pl


def kernel(center_ids, neigh_cell, neigh_drug, neigh_gene, gene_features, drug_features, cell_embed, W_gene, b_gene, W_drug, b_drug, att_w):
    raise NotImplementedError("write your pallas kernel here")


# speedup vs baseline: 1.9203x; 1.9203x over previous
import functools

import jax
import jax.numpy as jnp
from jax import lax
from jax.experimental import pallas as pl
from jax.experimental.pallas import tpu as pltpu
from jax.experimental.pallas import tpu_sc as plsc

B = 10000
S = 10
D = 128
NG = 100000
ND = 10000
NCELL = 1000
GFD = 128
DFD = 512

NCORE = 2
NSUB = 16
NW = NCORE * NSUB
BP = 10240
BPW = BP // NW
CH = 64
NCHUNK = BPW // CH
NIDX = CH * S // 128


def _sc_body(cidx_hbm, gidx_hbm, didx_hbm, xidx_hbm,
             gene_hbm, pdrug_hbm, cell_hbm,
             outc, outg, outd, outx,
             idx_v, rows_v, acc_v, cidx_v, crow_v, sem):
    wid = lax.axis_index("s") * NCORE + lax.axis_index("c")
    base = wid * BPW

    def chunk_body(k, carry):
        cbase = base + k * CH

        pltpu.sync_copy(cidx_hbm.at[pl.ds(cbase, CH)], cidx_v)
        pltpu.async_copy(gene_hbm.at[cidx_v], crow_v, sem).wait()
        pltpu.sync_copy(crow_v, outc.at[pl.ds(cbase, CH)])

        for idx_hbm, tab, outt in ((gidx_hbm, gene_hbm, outg),
                                   (didx_hbm, pdrug_hbm, outd),
                                   (xidx_hbm, cell_hbm, outx)):
            for j in range(NIDX):
                pltpu.sync_copy(idx_hbm.at[pl.ds(cbase * S + j * 128, 128)],
                                idx_v.at[j])
            handles = [
                pltpu.async_copy(tab.at[idx_v.at[j]],
                                 rows_v.at[pl.ds(j * 128, 128)], sem)
                for j in range(NIDX)
            ]
            for h in handles:
                h.wait()

            def red(c, acc):
                r0 = c * S
                for d in range(D // 16):
                    v = rows_v[r0, pl.ds(d * 16, 16)]
                    for s in range(1, S):
                        v = v + rows_v[r0 + s, pl.ds(d * 16, 16)]
                    acc_v[c, pl.ds(d * 16, 16)] = v
                return acc

            lax.fori_loop(0, CH, red, 0)
            pltpu.sync_copy(acc_v, outt.at[pl.ds(cbase, CH)])
        return carry

    lax.fori_loop(0, NCHUNK, chunk_body, 0)


@functools.cache
def _sc_gather_fn():
    return pl.kernel(
        _sc_body,
        out_type=(
            jax.ShapeDtypeStruct((BP, D), jnp.float32),
            jax.ShapeDtypeStruct((BP, D), jnp.float32),
            jax.ShapeDtypeStruct((BP, D), jnp.float32),
            jax.ShapeDtypeStruct((BP, D), jnp.float32),
        ),
        mesh=plsc.VectorSubcoreMesh(core_axis_name="c", subcore_axis_name="s",
                                    num_cores=NCORE, num_subcores=NSUB),
        scratch_types=(
            pltpu.VMEM((NIDX, 128), jnp.int32),
            pltpu.VMEM((CH * S, D), jnp.float32),
            pltpu.VMEM((CH, D), jnp.float32),
            pltpu.VMEM((CH,), jnp.int32),
            pltpu.VMEM((CH, D), jnp.float32),
            pltpu.SemaphoreType.DMA,
        ),
    )


def _proj_body(x_ref, w_ref, b_ref, o_ref):
    o_ref[...] = (jnp.dot(x_ref[...], w_ref[...],
                          preferred_element_type=jnp.float32) + b_ref[...])


_proj_drug = pl.pallas_call(
    _proj_body,
    grid=(10,),
    in_specs=[pl.BlockSpec((ND // 10, DFD), lambda i: (i, 0)),
              pl.BlockSpec((DFD, D), lambda i: (0, 0)),
              pl.BlockSpec((1, D), lambda i: (0, 0))],
    out_specs=pl.BlockSpec((ND // 10, D), lambda i: (i, 0)),
    out_shape=jax.ShapeDtypeStruct((ND, D), jnp.float32),
)


def _att_body(c_ref, g_ref, dr_ref, x_ref, w_ref, b_ref, a_ref, o_ref):
    inv_s = 1.0 / S
    h = (jnp.dot(c_ref[...], w_ref[...],
                 preferred_element_type=jnp.float32) + b_ref[...])
    ag = (jnp.dot(g_ref[...] * inv_s, w_ref[...],
                  preferred_element_type=jnp.float32) + b_ref[...])
    ad = dr_ref[...] * inv_s
    ax = x_ref[...] * inv_s
    a1 = a_ref[0:1, :]
    a2 = a_ref[1:2, :]

    base = jnp.sum(h * a1, axis=1, keepdims=True)

    def lrelu(v):
        return jnp.where(v >= 0, v, 0.01 * v)

    s0 = lrelu(base + jnp.sum(h * a2, axis=1, keepdims=True))
    s1 = lrelu(base + jnp.sum(ax * a2, axis=1, keepdims=True))
    s2 = lrelu(base + jnp.sum(ad * a2, axis=1, keepdims=True))
    s3 = lrelu(base + jnp.sum(ag * a2, axis=1, keepdims=True))
    m = jnp.maximum(jnp.maximum(s0, s1), jnp.maximum(s2, s3))
    e0 = jnp.exp(s0 - m)
    e1 = jnp.exp(s1 - m)
    e2 = jnp.exp(s2 - m)
    e3 = jnp.exp(s3 - m)
    z = e0 + e1 + e2 + e3
    o_ref[...] = (e0 * h + e1 * ax + e2 * ad + e3 * ag) / z


_att = pl.pallas_call(
    _att_body,
    grid=(10,),
    in_specs=[pl.BlockSpec((BP // 10, D), lambda i: (i, 0)),
              pl.BlockSpec((BP // 10, D), lambda i: (i, 0)),
              pl.BlockSpec((BP // 10, D), lambda i: (i, 0)),
              pl.BlockSpec((BP // 10, D), lambda i: (i, 0)),
              pl.BlockSpec((D, D), lambda i: (0, 0)),
              pl.BlockSpec((1, D), lambda i: (0, 0)),
              pl.BlockSpec((2, D), lambda i: (0, 0))],
    out_specs=pl.BlockSpec((BP // 10, D), lambda i: (i, 0)),
    out_shape=jax.ShapeDtypeStruct((BP, D), jnp.float32),
)


def kernel(center_ids, neigh_cell, neigh_drug, neigh_gene,
           gene_features, drug_features, cell_embed,
           W_gene, b_gene, W_drug, b_drug, att_w):
    pad = BP - B
    ci = jnp.pad(center_ids.astype(jnp.int32), (0, pad))
    ng = jnp.pad(neigh_gene.astype(jnp.int32).reshape(-1), (0, pad * S))
    nd = jnp.pad(neigh_drug.astype(jnp.int32).reshape(-1), (0, pad * S))
    nx = jnp.pad(neigh_cell.astype(jnp.int32).reshape(-1), (0, pad * S))

    pdrug = _proj_drug(drug_features, W_drug, b_drug.reshape(1, D))
    outc, outg, outd, outx = _sc_gather_fn()(ci, ng, nd, nx,
                                             gene_features, pdrug, cell_embed)
    out = _att(outc, outg, outd, outx,
               W_gene, b_gene.reshape(1, D), att_w.reshape(2, D))
    return out[:B]


# speedup vs baseline: 3.0936x; 1.6110x over previous
import functools

import jax
import jax.numpy as jnp
from jax import lax
from jax.experimental import pallas as pl
from jax.experimental.pallas import tpu as pltpu
from jax.experimental.pallas import tpu_sc as plsc

B = 10000
S = 10
D = 128
NG = 100000
ND = 10000
NCELL = 1000
GFD = 128
DFD = 512

NCORE = 2
NSUB = 16
NW = NCORE * NSUB
BP = 10240
BPW = BP // NW
CH = 32
NCH = BPW // CH
IDXW = CH * S
NSUP = NCH // 2


def _sc_body(cidx_hbm, gidx_hbm, didx_hbm, xidx_hbm,
             gene_hbm, pdrug_hbm, cell_hbm,
             outc, outg, outd, outx,
             gidx_v, didx_v, xidx_v, cidx_v,
             rows0, rows1, accg, accd, accx, cb0, cb1,
             semg0, semg1, semc, semog, semod, semox, semoc):
    wid = lax.axis_index("s") * NCORE + lax.axis_index("c")
    base = wid * BPW

    pltpu.sync_copy(gidx_hbm.at[wid], gidx_v)
    pltpu.sync_copy(didx_hbm.at[wid], didx_v)
    pltpu.sync_copy(xidx_hbm.at[wid], xidx_v)
    pltpu.sync_copy(cidx_hbm.at[wid], cidx_v)

    rows = (rows0, rows1)
    semg = (semg0, semg1)
    cbuf = (cb0, cb1)

    def issue_item(idx_v, tab, k, p):
        off = k * IDXW
        pltpu.async_copy(tab.at[idx_v.at[pl.ds(off, 128)]],
                         rows[p].at[pl.ds(0, 128)], semg[p])
        pltpu.async_copy(tab.at[idx_v.at[pl.ds(off + 128, 128)]],
                         rows[p].at[pl.ds(128, 128)], semg[p])
        pltpu.async_copy(tab.at[idx_v.at[pl.ds(off + 256, 64)]],
                         rows[p].at[pl.ds(256, 64)], semg[p])

    def wait_rows(p):
        pltpu.make_async_copy(gene_hbm.at[pl.ds(0, IDXW)], rows[p],
                              semg[p]).wait()

    def reduce_into(buf, acc):
        def red(c, carry):
            r0 = c * S
            for d in range(D // 16):
                v = buf[r0, pl.ds(d * 16, 16)]
                for s in range(1, S):
                    v = v + buf[r0 + s, pl.ds(d * 16, 16)]
                acc[c, pl.ds(d * 16, 16)] = v
            return carry
        lax.fori_loop(0, CH, red, 0)

    def drain_out(acc, outt, sem):
        pltpu.make_async_copy(acc, outt.at[pl.ds(0, CH)], sem).wait()

    issue_item(gidx_v, gene_hbm, 0, 0)
    pltpu.async_copy(gene_hbm.at[cidx_v.at[pl.ds(0, CH)]], cb0, semc)

    def sup_body(j, carry):
        k0 = 2 * j
        k1 = 2 * j + 1
        issue_item(didx_v, pdrug_hbm, k0, 1)
        wait_rows(0)

        @pl.when(j > 0)
        def _():
            drain_out(accg, outg, semog)
        reduce_into(rows[0], accg)
        pltpu.async_copy(accg, outg.at[pl.ds(base + k0 * CH, CH)], semog)

        issue_item(xidx_v, cell_hbm, k0, 0)
        wait_rows(1)

        @pl.when(j > 0)
        def _():
            drain_out(accd, outd, semod)
        reduce_into(rows[1], accd)
        pltpu.async_copy(accd, outd.at[pl.ds(base + k0 * CH, CH)], semod)

        issue_item(gidx_v, gene_hbm, k1, 1)
        wait_rows(0)

        @pl.when(j > 0)
        def _():
            drain_out(accx, outx, semox)
        reduce_into(rows[0], accx)
        pltpu.async_copy(accx, outx.at[pl.ds(base + k0 * CH, CH)], semox)

        pltpu.make_async_copy(gene_hbm.at[pl.ds(0, CH)], cb0, semc).wait()

        @pl.when(j > 0)
        def _():
            pltpu.make_async_copy(cb1, outc.at[pl.ds(0, CH)], semoc).wait()
        pltpu.async_copy(gene_hbm.at[cidx_v.at[pl.ds(k1 * CH, CH)]],
                         cb1, semc)
        pltpu.async_copy(cb0, outc.at[pl.ds(base + k0 * CH, CH)], semoc)

        issue_item(didx_v, pdrug_hbm, k1, 0)
        wait_rows(1)
        drain_out(accg, outg, semog)
        reduce_into(rows[1], accg)
        pltpu.async_copy(accg, outg.at[pl.ds(base + k1 * CH, CH)], semog)

        issue_item(xidx_v, cell_hbm, k1, 1)
        wait_rows(0)
        drain_out(accd, outd, semod)
        reduce_into(rows[0], accd)
        pltpu.async_copy(accd, outd.at[pl.ds(base + k1 * CH, CH)], semod)

        @pl.when(j < NSUP - 1)
        def _():
            issue_item(gidx_v, gene_hbm, k1 + 1, 0)
        wait_rows(1)
        drain_out(accx, outx, semox)
        reduce_into(rows[1], accx)
        pltpu.async_copy(accx, outx.at[pl.ds(base + k1 * CH, CH)], semox)

        pltpu.make_async_copy(gene_hbm.at[pl.ds(0, CH)], cb1, semc).wait()
        pltpu.make_async_copy(cb0, outc.at[pl.ds(0, CH)], semoc).wait()

        @pl.when(j < NSUP - 1)
        def _():
            pltpu.async_copy(gene_hbm.at[cidx_v.at[pl.ds((k1 + 1) * CH, CH)]],
                             cb0, semc)
        pltpu.async_copy(cb1, outc.at[pl.ds(base + k1 * CH, CH)], semoc)
        return carry

    lax.fori_loop(0, NSUP, sup_body, 0)

    drain_out(accg, outg, semog)
    drain_out(accd, outd, semod)
    drain_out(accx, outx, semox)
    pltpu.make_async_copy(cb1, outc.at[pl.ds(0, CH)], semoc).wait()


@functools.cache
def _sc_gather_fn():
    return pl.kernel(
        _sc_body,
        out_type=(
            jax.ShapeDtypeStruct((BP, D), jnp.float32),
            jax.ShapeDtypeStruct((BP, D), jnp.float32),
            jax.ShapeDtypeStruct((BP, D), jnp.float32),
            jax.ShapeDtypeStruct((BP, D), jnp.float32),
        ),
        mesh=plsc.VectorSubcoreMesh(core_axis_name="c", subcore_axis_name="s",
                                    num_cores=NCORE, num_subcores=NSUB),
        scratch_types=(
            pltpu.VMEM((BPW * S,), jnp.int32),
            pltpu.VMEM((BPW * S,), jnp.int32),
            pltpu.VMEM((BPW * S,), jnp.int32),
            pltpu.VMEM((BPW,), jnp.int32),
            pltpu.VMEM((IDXW, D), jnp.float32),
            pltpu.VMEM((IDXW, D), jnp.float32),
            pltpu.VMEM((CH, D), jnp.float32),
            pltpu.VMEM((CH, D), jnp.float32),
            pltpu.VMEM((CH, D), jnp.float32),
            pltpu.VMEM((CH, D), jnp.float32),
            pltpu.VMEM((CH, D), jnp.float32),
            pltpu.SemaphoreType.DMA,
            pltpu.SemaphoreType.DMA,
            pltpu.SemaphoreType.DMA,
            pltpu.SemaphoreType.DMA,
            pltpu.SemaphoreType.DMA,
            pltpu.SemaphoreType.DMA,
            pltpu.SemaphoreType.DMA,
        ),
    )


def _proj_body(x_ref, w_ref, b_ref, o_ref):
    o_ref[...] = (jnp.dot(x_ref[...], w_ref[...],
                          preferred_element_type=jnp.float32) + b_ref[...])


_proj_drug = pl.pallas_call(
    _proj_body,
    grid=(10,),
    in_specs=[pl.BlockSpec((ND // 10, DFD), lambda i: (i, 0)),
              pl.BlockSpec((DFD, D), lambda i: (0, 0)),
              pl.BlockSpec((1, D), lambda i: (0, 0))],
    out_specs=pl.BlockSpec((ND // 10, D), lambda i: (i, 0)),
    out_shape=jax.ShapeDtypeStruct((ND, D), jnp.float32),
)


def _att_body(c_ref, g_ref, dr_ref, x_ref, w_ref, b_ref, a_ref, o_ref):
    inv_s = 1.0 / S
    h = (jnp.dot(c_ref[...], w_ref[...],
                 preferred_element_type=jnp.float32) + b_ref[...])
    ag = (jnp.dot(g_ref[...] * inv_s, w_ref[...],
                  preferred_element_type=jnp.float32) + b_ref[...])
    ad = dr_ref[...] * inv_s
    ax = x_ref[...] * inv_s
    a1 = a_ref[0:1, :]
    a2 = a_ref[1:2, :]

    base = jnp.sum(h * a1, axis=1, keepdims=True)

    def lrelu(v):
        return jnp.where(v >= 0, v, 0.01 * v)

    s0 = lrelu(base + jnp.sum(h * a2, axis=1, keepdims=True))
    s1 = lrelu(base + jnp.sum(ax * a2, axis=1, keepdims=True))
    s2 = lrelu(base + jnp.sum(ad * a2, axis=1, keepdims=True))
    s3 = lrelu(base + jnp.sum(ag * a2, axis=1, keepdims=True))
    m = jnp.maximum(jnp.maximum(s0, s1), jnp.maximum(s2, s3))
    e0 = jnp.exp(s0 - m)
    e1 = jnp.exp(s1 - m)
    e2 = jnp.exp(s2 - m)
    e3 = jnp.exp(s3 - m)
    z = e0 + e1 + e2 + e3
    o_ref[...] = (e0 * h + e1 * ax + e2 * ad + e3 * ag) / z


_att = pl.pallas_call(
    _att_body,
    grid=(10,),
    in_specs=[pl.BlockSpec((BP // 10, D), lambda i: (i, 0)),
              pl.BlockSpec((BP // 10, D), lambda i: (i, 0)),
              pl.BlockSpec((BP // 10, D), lambda i: (i, 0)),
              pl.BlockSpec((BP // 10, D), lambda i: (i, 0)),
              pl.BlockSpec((D, D), lambda i: (0, 0)),
              pl.BlockSpec((1, D), lambda i: (0, 0)),
              pl.BlockSpec((2, D), lambda i: (0, 0))],
    out_specs=pl.BlockSpec((BP // 10, D), lambda i: (i, 0)),
    out_shape=jax.ShapeDtypeStruct((BP, D), jnp.float32),
)


def kernel(center_ids, neigh_cell, neigh_drug, neigh_gene,
           gene_features, drug_features, cell_embed,
           W_gene, b_gene, W_drug, b_drug, att_w):
    pad = BP - B
    ci = jnp.pad(center_ids.astype(jnp.int32), (0, pad)).reshape(NW, BPW)
    ng = jnp.pad(neigh_gene.astype(jnp.int32).reshape(-1),
                 (0, pad * S)).reshape(NW, BPW * S)
    nd = jnp.pad(neigh_drug.astype(jnp.int32).reshape(-1),
                 (0, pad * S)).reshape(NW, BPW * S)
    nx = jnp.pad(neigh_cell.astype(jnp.int32).reshape(-1),
                 (0, pad * S)).reshape(NW, BPW * S)

    pdrug = _proj_drug(drug_features, W_drug, b_drug.reshape(1, D))
    outc, outg, outd, outx = _sc_gather_fn()(ci, ng, nd, nx,
                                             gene_features, pdrug, cell_embed)
    out = _att(outc, outg, outd, outx,
               W_gene, b_gene.reshape(1, D), att_w.reshape(2, D))
    return out[:B]


# speedup vs baseline: 3.4029x; 1.1000x over previous
import functools

import jax
import jax.numpy as jnp
from jax import lax
from jax.experimental import pallas as pl
from jax.experimental.pallas import tpu as pltpu
from jax.experimental.pallas import tpu_sc as plsc

B = 10000
S = 10
D = 128
NG = 100000
ND = 10000
NCELL = 1000
GFD = 128
DFD = 512

NCORE = 2
NSUB = 16
NW = NCORE * NSUB
BP = 10240
CH = 32
IDXW = CH * S
ROWS_C0 = 448
ROWS_C1 = 192
NSUP0 = ROWS_C0 // (2 * CH)
NSUP1 = ROWS_C1 // (2 * CH)
BPC = BP + ROWS_C0


def _sc_body(cidx_hbm, gidx_hbm, didx_hbm, xidx_hbm,
             gene_hbm, pdrug_hbm, cell_hbm,
             outc, outg, outd, outx,
             gidx_v, didx_v, xidx_v, cidx_v,
             rows0, rows1, accg, accd, accx, cb0, cb1,
             semg0, semg1, semc, semog, semod, semox, semoc):
    c = lax.axis_index("c")
    sub = lax.axis_index("s")
    base = jnp.where(c == 0, sub * ROWS_C0, NSUB * ROWS_C0 + sub * ROWS_C1)
    nsup = jnp.where(c == 0, NSUP0, NSUP1)

    pltpu.sync_copy(gidx_hbm.at[pl.ds(base * S, ROWS_C0 * S)], gidx_v)
    pltpu.sync_copy(didx_hbm.at[pl.ds(base * S, ROWS_C0 * S)], didx_v)
    pltpu.sync_copy(xidx_hbm.at[pl.ds(base * S, ROWS_C0 * S)], xidx_v)
    pltpu.sync_copy(cidx_hbm.at[pl.ds(base, ROWS_C0)], cidx_v)

    rows = (rows0, rows1)
    semg = (semg0, semg1)
    cbuf = (cb0, cb1)

    def issue_item(idx_v, tab, k, p):
        off = k * IDXW
        pltpu.async_copy(tab.at[idx_v.at[pl.ds(off, 128)]],
                         rows[p].at[pl.ds(0, 128)], semg[p])
        pltpu.async_copy(tab.at[idx_v.at[pl.ds(off + 128, 128)]],
                         rows[p].at[pl.ds(128, 128)], semg[p])
        pltpu.async_copy(tab.at[idx_v.at[pl.ds(off + 256, 64)]],
                         rows[p].at[pl.ds(256, 64)], semg[p])

    def wait_rows(p):
        pltpu.make_async_copy(gene_hbm.at[pl.ds(0, IDXW)], rows[p],
                              semg[p]).wait()

    def reduce_into(buf, acc):
        def red(c, carry):
            r0 = c * S
            for d in range(D // 16):
                v = buf[r0, pl.ds(d * 16, 16)]
                for s in range(1, S):
                    v = v + buf[r0 + s, pl.ds(d * 16, 16)]
                acc[c, pl.ds(d * 16, 16)] = v
            return carry
        lax.fori_loop(0, CH, red, 0)

    def drain_out(acc, outt, sem):
        pltpu.make_async_copy(acc, outt.at[pl.ds(0, CH)], sem).wait()

    issue_item(gidx_v, gene_hbm, 0, 0)
    pltpu.async_copy(gene_hbm.at[cidx_v.at[pl.ds(0, CH)]], cb0, semc)

    def sup_body(j, carry):
        k0 = 2 * j
        k1 = 2 * j + 1
        issue_item(didx_v, pdrug_hbm, k0, 1)
        wait_rows(0)

        @pl.when(j > 0)
        def _():
            drain_out(accg, outg, semog)
        reduce_into(rows[0], accg)
        pltpu.async_copy(accg, outg.at[pl.ds(base + k0 * CH, CH)], semog)

        issue_item(xidx_v, cell_hbm, k0, 0)
        wait_rows(1)

        @pl.when(j > 0)
        def _():
            drain_out(accd, outd, semod)
        reduce_into(rows[1], accd)
        pltpu.async_copy(accd, outd.at[pl.ds(base + k0 * CH, CH)], semod)

        issue_item(gidx_v, gene_hbm, k1, 1)
        wait_rows(0)

        @pl.when(j > 0)
        def _():
            drain_out(accx, outx, semox)
        reduce_into(rows[0], accx)
        pltpu.async_copy(accx, outx.at[pl.ds(base + k0 * CH, CH)], semox)

        pltpu.make_async_copy(gene_hbm.at[pl.ds(0, CH)], cb0, semc).wait()

        @pl.when(j > 0)
        def _():
            pltpu.make_async_copy(cb1, outc.at[pl.ds(0, CH)], semoc).wait()
        pltpu.async_copy(gene_hbm.at[cidx_v.at[pl.ds(k1 * CH, CH)]],
                         cb1, semc)
        pltpu.async_copy(cb0, outc.at[pl.ds(base + k0 * CH, CH)], semoc)

        issue_item(didx_v, pdrug_hbm, k1, 0)
        wait_rows(1)
        drain_out(accg, outg, semog)
        reduce_into(rows[1], accg)
        pltpu.async_copy(accg, outg.at[pl.ds(base + k1 * CH, CH)], semog)

        issue_item(xidx_v, cell_hbm, k1, 1)
        wait_rows(0)
        drain_out(accd, outd, semod)
        reduce_into(rows[0], accd)
        pltpu.async_copy(accd, outd.at[pl.ds(base + k1 * CH, CH)], semod)

        @pl.when(j < nsup - 1)
        def _():
            issue_item(gidx_v, gene_hbm, k1 + 1, 0)
        wait_rows(1)
        drain_out(accx, outx, semox)
        reduce_into(rows[1], accx)
        pltpu.async_copy(accx, outx.at[pl.ds(base + k1 * CH, CH)], semox)

        pltpu.make_async_copy(gene_hbm.at[pl.ds(0, CH)], cb1, semc).wait()
        pltpu.make_async_copy(cb0, outc.at[pl.ds(0, CH)], semoc).wait()

        @pl.when(j < nsup - 1)
        def _():
            pltpu.async_copy(gene_hbm.at[cidx_v.at[pl.ds((k1 + 1) * CH, CH)]],
                             cb0, semc)
        pltpu.async_copy(cb1, outc.at[pl.ds(base + k1 * CH, CH)], semoc)
        return carry

    lax.fori_loop(0, nsup, sup_body, 0)

    drain_out(accg, outg, semog)
    drain_out(accd, outd, semod)
    drain_out(accx, outx, semox)
    pltpu.make_async_copy(cb1, outc.at[pl.ds(0, CH)], semoc).wait()


@functools.cache
def _sc_gather_fn():
    return pl.kernel(
        _sc_body,
        out_type=(
            jax.ShapeDtypeStruct((BP, D), jnp.float32),
            jax.ShapeDtypeStruct((BP, D), jnp.float32),
            jax.ShapeDtypeStruct((BP, D), jnp.float32),
            jax.ShapeDtypeStruct((BP, D), jnp.float32),
        ),
        mesh=plsc.VectorSubcoreMesh(core_axis_name="c", subcore_axis_name="s",
                                    num_cores=NCORE, num_subcores=NSUB),
        scratch_types=(
            pltpu.VMEM((ROWS_C0 * S,), jnp.int32),
            pltpu.VMEM((ROWS_C0 * S,), jnp.int32),
            pltpu.VMEM((ROWS_C0 * S,), jnp.int32),
            pltpu.VMEM((ROWS_C0,), jnp.int32),
            pltpu.VMEM((IDXW, D), jnp.float32),
            pltpu.VMEM((IDXW, D), jnp.float32),
            pltpu.VMEM((CH, D), jnp.float32),
            pltpu.VMEM((CH, D), jnp.float32),
            pltpu.VMEM((CH, D), jnp.float32),
            pltpu.VMEM((CH, D), jnp.float32),
            pltpu.VMEM((CH, D), jnp.float32),
            pltpu.SemaphoreType.DMA,
            pltpu.SemaphoreType.DMA,
            pltpu.SemaphoreType.DMA,
            pltpu.SemaphoreType.DMA,
            pltpu.SemaphoreType.DMA,
            pltpu.SemaphoreType.DMA,
            pltpu.SemaphoreType.DMA,
        ),
    )


def _proj_body(x_ref, w_ref, b_ref, o_ref):
    o_ref[...] = (jnp.dot(x_ref[...], w_ref[...],
                          preferred_element_type=jnp.float32) + b_ref[...])


_proj_drug = pl.pallas_call(
    _proj_body,
    grid=(10,),
    in_specs=[pl.BlockSpec((ND // 10, DFD), lambda i: (i, 0)),
              pl.BlockSpec((DFD, D), lambda i: (0, 0)),
              pl.BlockSpec((1, D), lambda i: (0, 0))],
    out_specs=pl.BlockSpec((ND // 10, D), lambda i: (i, 0)),
    out_shape=jax.ShapeDtypeStruct((ND, D), jnp.float32),
)


def _att_body(c_ref, g_ref, dr_ref, x_ref, w_ref, b_ref, a_ref, o_ref):
    inv_s = 1.0 / S
    h = (jnp.dot(c_ref[...], w_ref[...],
                 preferred_element_type=jnp.float32) + b_ref[...])
    ag = (jnp.dot(g_ref[...] * inv_s, w_ref[...],
                  preferred_element_type=jnp.float32) + b_ref[...])
    ad = dr_ref[...] * inv_s
    ax = x_ref[...] * inv_s
    a1 = a_ref[0:1, :]
    a2 = a_ref[1:2, :]

    base = jnp.sum(h * a1, axis=1, keepdims=True)

    def lrelu(v):
        return jnp.where(v >= 0, v, 0.01 * v)

    s0 = lrelu(base + jnp.sum(h * a2, axis=1, keepdims=True))
    s1 = lrelu(base + jnp.sum(ax * a2, axis=1, keepdims=True))
    s2 = lrelu(base + jnp.sum(ad * a2, axis=1, keepdims=True))
    s3 = lrelu(base + jnp.sum(ag * a2, axis=1, keepdims=True))
    m = jnp.maximum(jnp.maximum(s0, s1), jnp.maximum(s2, s3))
    e0 = jnp.exp(s0 - m)
    e1 = jnp.exp(s1 - m)
    e2 = jnp.exp(s2 - m)
    e3 = jnp.exp(s3 - m)
    z = e0 + e1 + e2 + e3
    o_ref[...] = (e0 * h + e1 * ax + e2 * ad + e3 * ag) / z


_att = pl.pallas_call(
    _att_body,
    grid=(10,),
    in_specs=[pl.BlockSpec((BP // 10, D), lambda i: (i, 0)),
              pl.BlockSpec((BP // 10, D), lambda i: (i, 0)),
              pl.BlockSpec((BP // 10, D), lambda i: (i, 0)),
              pl.BlockSpec((BP // 10, D), lambda i: (i, 0)),
              pl.BlockSpec((D, D), lambda i: (0, 0)),
              pl.BlockSpec((1, D), lambda i: (0, 0)),
              pl.BlockSpec((2, D), lambda i: (0, 0))],
    out_specs=pl.BlockSpec((BP // 10, D), lambda i: (i, 0)),
    out_shape=jax.ShapeDtypeStruct((BP, D), jnp.float32),
)


def kernel(center_ids, neigh_cell, neigh_drug, neigh_gene,
           gene_features, drug_features, cell_embed,
           W_gene, b_gene, W_drug, b_drug, att_w):
    pad = BPC - B
    ci = jnp.pad(center_ids.astype(jnp.int32), (0, pad))
    ng = jnp.pad(neigh_gene.astype(jnp.int32).reshape(-1), (0, pad * S))
    nd = jnp.pad(neigh_drug.astype(jnp.int32).reshape(-1), (0, pad * S))
    nx = jnp.pad(neigh_cell.astype(jnp.int32).reshape(-1), (0, pad * S))

    pdrug = _proj_drug(drug_features, W_drug, b_drug.reshape(1, D))
    outc, outg, outd, outx = _sc_gather_fn()(ci, ng, nd, nx,
                                             gene_features, pdrug, cell_embed)
    out = _att(outc, outg, outd, outx,
               W_gene, b_gene.reshape(1, D), att_w.reshape(2, D))
    return out[:B]


# speedup vs baseline: 4.0180x; 1.1807x over previous
import functools

import jax
import jax.numpy as jnp
from jax import lax
from jax.experimental import pallas as pl
from jax.experimental.pallas import tpu as pltpu
from jax.experimental.pallas import tpu_sc as plsc

B = 10000
S = 10
D = 128
NG = 100000
ND = 10000
NCELL = 1000
GFD = 128
DFD = 512
DP = D // 2

NCORE = 2
NSUB = 16
NW = NCORE * NSUB
BP = 10240
CH = 32
IDXW = CH * S
ROWS_C0 = 480
ROWS_C1 = 160
NCH0 = ROWS_C0 // CH
NCH1 = ROWS_C1 // CH
BPC = BP + ROWS_C0
MASKHI = -65536


def _sc_body(cidx_hbm, gidx_hbm, didx_hbm, xidx_hbm,
             gene_hbm, pdrug_hbm, cell_hbm,
             outc, outg, outd, outx,
             gidx_v, didx_v, xidx_v, cidx_v,
             bf, b0, b1, accg, accd, accx, cb0, cb1,
             semf, semb0, semb1, semc, semog, semod, semox, semoc):
    core = lax.axis_index("c")
    sub = lax.axis_index("s")
    base = jnp.where(core == 0, sub * ROWS_C0,
                     NSUB * ROWS_C0 + sub * ROWS_C1)
    nch = jnp.where(core == 0, NCH0, NCH1)

    pltpu.sync_copy(gidx_hbm.at[pl.ds(base * S, ROWS_C0 * S)], gidx_v)
    pltpu.sync_copy(didx_hbm.at[pl.ds(base * S, ROWS_C0 * S)], didx_v)
    pltpu.sync_copy(xidx_hbm.at[pl.ds(base * S, ROWS_C0 * S)], xidx_v)
    pltpu.sync_copy(cidx_hbm.at[pl.ds(base, ROWS_C0)], cidx_v)

    def issue_item(idx_v, tab, k, buf, sem):
        off = k * IDXW
        w = buf.shape[1]
        pltpu.async_copy(tab.at[idx_v.at[pl.ds(off, 128)]],
                         buf.at[pl.ds(0, 128)], sem)
        pltpu.async_copy(tab.at[idx_v.at[pl.ds(off + 128, 128)]],
                         buf.at[pl.ds(128, 128)], sem)
        pltpu.async_copy(tab.at[idx_v.at[pl.ds(off + 256, 64)]],
                         buf.at[pl.ds(256, 64)], sem)

    def reduce_f32(buf, acc):
        def red(cc, carry):
            r0 = cc * S
            for dd in range(D // 16):
                v = buf[r0, pl.ds(dd * 16, 16)]
                for s in range(1, S):
                    v = v + buf[r0 + s, pl.ds(dd * 16, 16)]
                acc[cc, pl.ds(dd * 16, 16)] = v
            return carry
        lax.fori_loop(0, CH, red, 0)

    def reduce_packed(buf, acc):
        shift = jnp.full((16,), 65536, jnp.int32)
        mask = jnp.full((16,), MASKHI, jnp.int32)

        def red(cc, carry):
            r0 = cc * S
            for blk in range(DP // 16):
                v = buf[r0, pl.ds(blk * 16, 16)]
                se = lax.bitcast_convert_type(v * shift, jnp.float32)
                so = lax.bitcast_convert_type(v & mask, jnp.float32)
                for s in range(1, S):
                    v = buf[r0 + s, pl.ds(blk * 16, 16)]
                    se = se + lax.bitcast_convert_type(v * shift,
                                                       jnp.float32)
                    so = so + lax.bitcast_convert_type(v & mask,
                                                       jnp.float32)
                acc[cc, pl.ds(blk * 32, 16)] = se
                acc[cc, pl.ds(blk * 32 + 16, 16)] = so
            return carry
        lax.fori_loop(0, CH, red, 0)

    def drain_out(acc, outt, sem):
        pltpu.make_async_copy(acc, outt.at[pl.ds(0, CH)], sem).wait()

    issue_item(gidx_v, gene_hbm, 0, bf, semf)

    def chunk_body(k, carry):
        ob = base + k * CH
        issue_item(didx_v, pdrug_hbm, k, b0, semb0)
        pltpu.make_async_copy(gene_hbm.at[pl.ds(0, IDXW)], bf, semf).wait()

        @pl.when(k > 0)
        def _():
            drain_out(accg, outg, semog)
        reduce_f32(bf, accg)
        pltpu.async_copy(accg, outg.at[pl.ds(ob, CH)], semog)

        issue_item(xidx_v, cell_hbm, k, b1, semb1)
        pltpu.make_async_copy(pdrug_hbm.at[pl.ds(0, IDXW)], b0,
                              semb0).wait()

        @pl.when(k > 0)
        def _():
            drain_out(accd, outd, semod)
        reduce_packed(b0, accd)
        pltpu.async_copy(accd, outd.at[pl.ds(ob, CH)], semod)

        @pl.when(k >= 2)
        def _():
            pltpu.make_async_copy(cb0, outc.at[pl.ds(0, CH)], semoc).wait()
        pe = (k % 2) == 0
        po = (k % 2) == 1

        @pl.when(pe)
        def _():
            pltpu.async_copy(gene_hbm.at[cidx_v.at[pl.ds(k * CH, CH)]],
                             cb0, semc)

        @pl.when(po)
        def _():
            pltpu.async_copy(gene_hbm.at[cidx_v.at[pl.ds(k * CH, CH)]],
                             cb1, semc)
        pltpu.make_async_copy(cell_hbm.at[pl.ds(0, IDXW)], b1,
                              semb1).wait()

        @pl.when(k > 0)
        def _():
            drain_out(accx, outx, semox)
        reduce_packed(b1, accx)
        pltpu.async_copy(accx, outx.at[pl.ds(ob, CH)], semox)

        @pl.when(k < nch - 1)
        def _():
            issue_item(gidx_v, gene_hbm, k + 1, bf, semf)
        pltpu.make_async_copy(gene_hbm.at[pl.ds(0, CH)], cb0, semc).wait()

        @pl.when(pe)
        def _():
            pltpu.async_copy(cb0, outc.at[pl.ds(ob, CH)], semoc)

        @pl.when(po)
        def _():
            pltpu.async_copy(cb1, outc.at[pl.ds(ob, CH)], semoc)
        return carry

    lax.fori_loop(0, nch, chunk_body, 0)

    drain_out(accg, outg, semog)
    drain_out(accd, outd, semod)
    drain_out(accx, outx, semox)
    pltpu.make_async_copy(cb0, outc.at[pl.ds(0, CH)], semoc).wait()
    pltpu.make_async_copy(cb1, outc.at[pl.ds(0, CH)], semoc).wait()


@functools.cache
def _sc_gather_fn():
    return pl.kernel(
        _sc_body,
        out_type=(
            jax.ShapeDtypeStruct((BP, D), jnp.float32),
            jax.ShapeDtypeStruct((BP, D), jnp.float32),
            jax.ShapeDtypeStruct((BP, D), jnp.float32),
            jax.ShapeDtypeStruct((BP, D), jnp.float32),
        ),
        mesh=plsc.VectorSubcoreMesh(core_axis_name="c", subcore_axis_name="s",
                                    num_cores=NCORE, num_subcores=NSUB),
        compiler_params=pltpu.CompilerParams(use_tc_tiling_on_sc=False),
        scratch_types=(
            pltpu.VMEM((ROWS_C0 * S,), jnp.int32),
            pltpu.VMEM((ROWS_C0 * S,), jnp.int32),
            pltpu.VMEM((ROWS_C0 * S,), jnp.int32),
            pltpu.VMEM((ROWS_C0,), jnp.int32),
            pltpu.VMEM((IDXW, D), jnp.float32),
            pltpu.VMEM((IDXW, DP), jnp.int32),
            pltpu.VMEM((IDXW, DP), jnp.int32),
            pltpu.VMEM((CH, D), jnp.float32),
            pltpu.VMEM((CH, D), jnp.float32),
            pltpu.VMEM((CH, D), jnp.float32),
            pltpu.VMEM((CH, D), jnp.float32),
            pltpu.VMEM((CH, D), jnp.float32),
            pltpu.SemaphoreType.DMA,
            pltpu.SemaphoreType.DMA,
            pltpu.SemaphoreType.DMA,
            pltpu.SemaphoreType.DMA,
            pltpu.SemaphoreType.DMA,
            pltpu.SemaphoreType.DMA,
            pltpu.SemaphoreType.DMA,
            pltpu.SemaphoreType.DMA,
        ),
    )


def _proj_body(x_ref, w_ref, b_ref, o_ref):
    o_ref[...] = (jnp.dot(x_ref[...], w_ref[...],
                          preferred_element_type=jnp.float32)
                  + b_ref[...]).astype(jnp.bfloat16)


_proj_drug = pl.pallas_call(
    _proj_body,
    grid=(5,),
    in_specs=[pl.BlockSpec((ND // 5, DFD), lambda i: (i, 0)),
              pl.BlockSpec((DFD, D), lambda i: (0, 0)),
              pl.BlockSpec((1, D), lambda i: (0, 0))],
    out_specs=pl.BlockSpec((ND // 5, D), lambda i: (i, 0)),
    out_shape=jax.ShapeDtypeStruct((ND, D), jnp.bfloat16),
)


def _att_body(c_ref, g_ref, dr_ref, x_ref, w_ref, p_ref, b_ref, a_ref,
              o_ref):
    inv_s = 1.0 / S
    h = (jnp.dot(c_ref[...], w_ref[...],
                 preferred_element_type=jnp.float32) + b_ref[...])
    ag = (jnp.dot(g_ref[...] * inv_s, w_ref[...],
                  preferred_element_type=jnp.float32) + b_ref[...])
    ad = jnp.dot(dr_ref[...] * inv_s, p_ref[...],
                 preferred_element_type=jnp.float32)
    ax = jnp.dot(x_ref[...] * inv_s, p_ref[...],
                 preferred_element_type=jnp.float32)
    a1 = a_ref[0:1, :]
    a2 = a_ref[1:2, :]

    base = jnp.sum(h * a1, axis=1, keepdims=True)

    def lrelu(v):
        return jnp.where(v >= 0, v, 0.01 * v)

    s0 = lrelu(base + jnp.sum(h * a2, axis=1, keepdims=True))
    s1 = lrelu(base + jnp.sum(ax * a2, axis=1, keepdims=True))
    s2 = lrelu(base + jnp.sum(ad * a2, axis=1, keepdims=True))
    s3 = lrelu(base + jnp.sum(ag * a2, axis=1, keepdims=True))
    m = jnp.maximum(jnp.maximum(s0, s1), jnp.maximum(s2, s3))
    e0 = jnp.exp(s0 - m)
    e1 = jnp.exp(s1 - m)
    e2 = jnp.exp(s2 - m)
    e3 = jnp.exp(s3 - m)
    z = e0 + e1 + e2 + e3
    o_ref[...] = (e0 * h + e1 * ax + e2 * ad + e3 * ag) / z


_att = pl.pallas_call(
    _att_body,
    grid=(10,),
    in_specs=[pl.BlockSpec((B // 10, D), lambda i: (i, 0)),
              pl.BlockSpec((B // 10, D), lambda i: (i, 0)),
              pl.BlockSpec((B // 10, D), lambda i: (i, 0)),
              pl.BlockSpec((B // 10, D), lambda i: (i, 0)),
              pl.BlockSpec((D, D), lambda i: (0, 0)),
              pl.BlockSpec((D, D), lambda i: (0, 0)),
              pl.BlockSpec((1, D), lambda i: (0, 0)),
              pl.BlockSpec((2, D), lambda i: (0, 0))],
    out_specs=pl.BlockSpec((B // 10, D), lambda i: (i, 0)),
    out_shape=jax.ShapeDtypeStruct((B, D), jnp.float32),
)


def _perm_matrix():
    j = jnp.arange(D)
    blk, r = j // 32, j % 32
    src = 32 * blk + jnp.where(r < 16, 2 * r, 2 * (r - 16) + 1)
    return (jnp.arange(D)[None, :] == src[:, None]).astype(jnp.float32)


def kernel(center_ids, neigh_cell, neigh_drug, neigh_gene,
           gene_features, drug_features, cell_embed,
           W_gene, b_gene, W_drug, b_drug, att_w):
    pad = BPC - B
    ci = jnp.pad(center_ids.astype(jnp.int32), (0, pad))
    ng = jnp.pad(neigh_gene.astype(jnp.int32).reshape(-1), (0, pad * S))
    nd = jnp.pad(neigh_drug.astype(jnp.int32).reshape(-1), (0, pad * S))
    nx = jnp.pad(neigh_cell.astype(jnp.int32).reshape(-1), (0, pad * S))

    pdrug = _proj_drug(drug_features, W_drug, b_drug.reshape(1, D))
    pdrug_i32 = lax.bitcast_convert_type(pdrug.reshape(ND, DP, 2),
                                         jnp.int32)
    cell_i32 = lax.bitcast_convert_type(
        cell_embed.astype(jnp.bfloat16).reshape(NCELL, DP, 2), jnp.int32)

    outc, outg, outd, outx = _sc_gather_fn()(ci, ng, nd, nx,
                                             gene_features, pdrug_i32,
                                             cell_i32)
    return _att(outc, outg, outd, outx,
                W_gene, _perm_matrix(), b_gene.reshape(1, D),
                att_w.reshape(2, D))


# speedup vs baseline: 4.0197x; 1.0004x over previous
import functools

import jax
import jax.numpy as jnp
from jax import lax
from jax.experimental import pallas as pl
from jax.experimental.pallas import tpu as pltpu
from jax.experimental.pallas import tpu_sc as plsc

B = 10000
S = 10
D = 128
NG = 100000
ND = 10000
NCELL = 1000
GFD = 128
DFD = 512
DP = D // 2

NCORE = 2
NSUB = 16
NW = NCORE * NSUB
BP = 10240
CH = 32
IDXW = CH * S
ROWS_C0 = 480
ROWS_C1 = 160
NCH0 = ROWS_C0 // CH
NCH1 = ROWS_C1 // CH
BPC = BP + ROWS_C0
MASKHI = -65536


def _sc_body(cidx_hbm, gidx_hbm, didx_hbm, xidx_hbm,
             gene_hbm, pdrug_hbm, cell_hbm,
             outc, outg, outd, outx,
             gidx_v, didx_v, xidx_v, cidx_v,
             bf, b0, b1, accg, accd, accx, cb0, cb1,
             semf, semb0, semb1, semc, semog, semod, semox, semoc):
    core = lax.axis_index("c")
    sub = lax.axis_index("s")
    base = jnp.where(core == 0, sub * ROWS_C0,
                     NSUB * ROWS_C0 + sub * ROWS_C1)
    nch = jnp.where(core == 0, NCH0, NCH1)

    pltpu.sync_copy(gidx_hbm.at[pl.ds(base * S, ROWS_C0 * S)], gidx_v)
    pltpu.sync_copy(didx_hbm.at[pl.ds(base * S, ROWS_C0 * S)], didx_v)
    pltpu.sync_copy(xidx_hbm.at[pl.ds(base * S, ROWS_C0 * S)], xidx_v)
    pltpu.sync_copy(cidx_hbm.at[pl.ds(base, ROWS_C0)], cidx_v)

    def issue_item(idx_v, tab, k, buf, sem):
        pltpu.async_copy(tab.at[idx_v.at[pl.ds(k * IDXW, IDXW)]], buf, sem)

    def reduce_f32(buf, acc):
        def red(cc, carry):
            r0 = cc * S
            for dd in range(D // 16):
                v = buf[r0, pl.ds(dd * 16, 16)]
                for s in range(1, S):
                    v = v + buf[r0 + s, pl.ds(dd * 16, 16)]
                acc[cc, pl.ds(dd * 16, 16)] = v
            return carry
        lax.fori_loop(0, CH, red, 0)

    def reduce_packed(buf, acc):
        shift = jnp.full((16,), 65536, jnp.int32)
        mask = jnp.full((16,), MASKHI, jnp.int32)

        def red(cc, carry):
            r0 = cc * S
            for blk in range(DP // 16):
                v = buf[r0, pl.ds(blk * 16, 16)]
                se = lax.bitcast_convert_type(v * shift, jnp.float32)
                so = lax.bitcast_convert_type(v & mask, jnp.float32)
                for s in range(1, S):
                    v = buf[r0 + s, pl.ds(blk * 16, 16)]
                    se = se + lax.bitcast_convert_type(v * shift,
                                                       jnp.float32)
                    so = so + lax.bitcast_convert_type(v & mask,
                                                       jnp.float32)
                acc[cc, pl.ds(blk * 32, 16)] = se
                acc[cc, pl.ds(blk * 32 + 16, 16)] = so
            return carry
        lax.fori_loop(0, CH, red, 0)

    def drain_out(acc, outt, sem):
        pltpu.make_async_copy(acc, outt.at[pl.ds(0, CH)], sem).wait()

    issue_item(gidx_v, gene_hbm, 0, bf, semf)

    def chunk_body(k, carry):
        ob = base + k * CH
        issue_item(didx_v, pdrug_hbm, k, b0, semb0)
        pltpu.make_async_copy(gene_hbm.at[pl.ds(0, IDXW)], bf, semf).wait()

        @pl.when(k > 0)
        def _():
            drain_out(accg, outg, semog)
        reduce_f32(bf, accg)
        pltpu.async_copy(accg, outg.at[pl.ds(ob, CH)], semog)

        issue_item(xidx_v, cell_hbm, k, b1, semb1)
        pltpu.make_async_copy(pdrug_hbm.at[pl.ds(0, IDXW)], b0,
                              semb0).wait()

        @pl.when(k > 0)
        def _():
            drain_out(accd, outd, semod)
        reduce_packed(b0, accd)
        pltpu.async_copy(accd, outd.at[pl.ds(ob, CH)], semod)

        @pl.when(k >= 2)
        def _():
            pltpu.make_async_copy(cb0, outc.at[pl.ds(0, CH)], semoc).wait()
        pe = (k % 2) == 0
        po = (k % 2) == 1

        @pl.when(pe)
        def _():
            pltpu.async_copy(gene_hbm.at[cidx_v.at[pl.ds(k * CH, CH)]],
                             cb0, semc)

        @pl.when(po)
        def _():
            pltpu.async_copy(gene_hbm.at[cidx_v.at[pl.ds(k * CH, CH)]],
                             cb1, semc)
        pltpu.make_async_copy(cell_hbm.at[pl.ds(0, IDXW)], b1,
                              semb1).wait()

        @pl.when(k > 0)
        def _():
            drain_out(accx, outx, semox)
        reduce_packed(b1, accx)
        pltpu.async_copy(accx, outx.at[pl.ds(ob, CH)], semox)

        @pl.when(k < nch - 1)
        def _():
            issue_item(gidx_v, gene_hbm, k + 1, bf, semf)
        pltpu.make_async_copy(gene_hbm.at[pl.ds(0, CH)], cb0, semc).wait()

        @pl.when(pe)
        def _():
            pltpu.async_copy(cb0, outc.at[pl.ds(ob, CH)], semoc)

        @pl.when(po)
        def _():
            pltpu.async_copy(cb1, outc.at[pl.ds(ob, CH)], semoc)
        return carry

    lax.fori_loop(0, nch, chunk_body, 0)

    drain_out(accg, outg, semog)
    drain_out(accd, outd, semod)
    drain_out(accx, outx, semox)
    pltpu.make_async_copy(cb0, outc.at[pl.ds(0, CH)], semoc).wait()
    pltpu.make_async_copy(cb1, outc.at[pl.ds(0, CH)], semoc).wait()


@functools.cache
def _sc_gather_fn():
    return pl.kernel(
        _sc_body,
        out_type=(
            jax.ShapeDtypeStruct((BP, D), jnp.float32),
            jax.ShapeDtypeStruct((BP, D), jnp.float32),
            jax.ShapeDtypeStruct((BP, D), jnp.float32),
            jax.ShapeDtypeStruct((BP, D), jnp.float32),
        ),
        mesh=plsc.VectorSubcoreMesh(core_axis_name="c", subcore_axis_name="s",
                                    num_cores=NCORE, num_subcores=NSUB),
        compiler_params=pltpu.CompilerParams(use_tc_tiling_on_sc=False),
        scratch_types=(
            pltpu.VMEM((ROWS_C0 * S,), jnp.int32),
            pltpu.VMEM((ROWS_C0 * S,), jnp.int32),
            pltpu.VMEM((ROWS_C0 * S,), jnp.int32),
            pltpu.VMEM((ROWS_C0,), jnp.int32),
            pltpu.VMEM((IDXW, D), jnp.float32),
            pltpu.VMEM((IDXW, DP), jnp.int32),
            pltpu.VMEM((IDXW, DP), jnp.int32),
            pltpu.VMEM((CH, D), jnp.float32),
            pltpu.VMEM((CH, D), jnp.float32),
            pltpu.VMEM((CH, D), jnp.float32),
            pltpu.VMEM((CH, D), jnp.float32),
            pltpu.VMEM((CH, D), jnp.float32),
            pltpu.SemaphoreType.DMA,
            pltpu.SemaphoreType.DMA,
            pltpu.SemaphoreType.DMA,
            pltpu.SemaphoreType.DMA,
            pltpu.SemaphoreType.DMA,
            pltpu.SemaphoreType.DMA,
            pltpu.SemaphoreType.DMA,
            pltpu.SemaphoreType.DMA,
        ),
    )


def _proj_body(x_ref, w_ref, b_ref, o_ref):
    o_ref[...] = (jnp.dot(x_ref[...], w_ref[...],
                          preferred_element_type=jnp.float32)
                  + b_ref[...]).astype(jnp.bfloat16)


_proj_drug = pl.pallas_call(
    _proj_body,
    grid=(5,),
    in_specs=[pl.BlockSpec((ND // 5, DFD), lambda i: (i, 0)),
              pl.BlockSpec((DFD, D), lambda i: (0, 0)),
              pl.BlockSpec((1, D), lambda i: (0, 0))],
    out_specs=pl.BlockSpec((ND // 5, D), lambda i: (i, 0)),
    out_shape=jax.ShapeDtypeStruct((ND, D), jnp.bfloat16),
)


def _att_body(c_ref, g_ref, dr_ref, x_ref, w_ref, p_ref, b_ref, a_ref,
              o_ref):
    inv_s = 1.0 / S
    h = (jnp.dot(c_ref[...], w_ref[...],
                 preferred_element_type=jnp.float32) + b_ref[...])
    ag = (jnp.dot(g_ref[...] * inv_s, w_ref[...],
                  preferred_element_type=jnp.float32) + b_ref[...])
    ad = jnp.dot(dr_ref[...] * inv_s, p_ref[...],
                 preferred_element_type=jnp.float32)
    ax = jnp.dot(x_ref[...] * inv_s, p_ref[...],
                 preferred_element_type=jnp.float32)
    a1 = a_ref[0:1, :]
    a2 = a_ref[1:2, :]

    base = jnp.sum(h * a1, axis=1, keepdims=True)

    def lrelu(v):
        return jnp.where(v >= 0, v, 0.01 * v)

    s0 = lrelu(base + jnp.sum(h * a2, axis=1, keepdims=True))
    s1 = lrelu(base + jnp.sum(ax * a2, axis=1, keepdims=True))
    s2 = lrelu(base + jnp.sum(ad * a2, axis=1, keepdims=True))
    s3 = lrelu(base + jnp.sum(ag * a2, axis=1, keepdims=True))
    m = jnp.maximum(jnp.maximum(s0, s1), jnp.maximum(s2, s3))
    e0 = jnp.exp(s0 - m)
    e1 = jnp.exp(s1 - m)
    e2 = jnp.exp(s2 - m)
    e3 = jnp.exp(s3 - m)
    z = e0 + e1 + e2 + e3
    o_ref[...] = (e0 * h + e1 * ax + e2 * ad + e3 * ag) / z


_att = pl.pallas_call(
    _att_body,
    grid=(10,),
    in_specs=[pl.BlockSpec((B // 10, D), lambda i: (i, 0)),
              pl.BlockSpec((B // 10, D), lambda i: (i, 0)),
              pl.BlockSpec((B // 10, D), lambda i: (i, 0)),
              pl.BlockSpec((B // 10, D), lambda i: (i, 0)),
              pl.BlockSpec((D, D), lambda i: (0, 0)),
              pl.BlockSpec((D, D), lambda i: (0, 0)),
              pl.BlockSpec((1, D), lambda i: (0, 0)),
              pl.BlockSpec((2, D), lambda i: (0, 0))],
    out_specs=pl.BlockSpec((B // 10, D), lambda i: (i, 0)),
    out_shape=jax.ShapeDtypeStruct((B, D), jnp.float32),
)


def _perm_matrix():
    j = jnp.arange(D)
    blk, r = j // 32, j % 32
    src = 32 * blk + jnp.where(r < 16, 2 * r, 2 * (r - 16) + 1)
    return (jnp.arange(D)[None, :] == src[:, None]).astype(jnp.float32)


def kernel(center_ids, neigh_cell, neigh_drug, neigh_gene,
           gene_features, drug_features, cell_embed,
           W_gene, b_gene, W_drug, b_drug, att_w):
    pad = BPC - B
    ci = jnp.pad(center_ids.astype(jnp.int32), (0, pad))
    ng = jnp.pad(neigh_gene.astype(jnp.int32).reshape(-1), (0, pad * S))
    nd = jnp.pad(neigh_drug.astype(jnp.int32).reshape(-1), (0, pad * S))
    nx = jnp.pad(neigh_cell.astype(jnp.int32).reshape(-1), (0, pad * S))

    pdrug = _proj_drug(drug_features, W_drug, b_drug.reshape(1, D))
    pdrug_i32 = lax.bitcast_convert_type(pdrug.reshape(ND, DP, 2),
                                         jnp.int32)
    cell_i32 = lax.bitcast_convert_type(
        cell_embed.astype(jnp.bfloat16).reshape(NCELL, DP, 2), jnp.int32)

    outc, outg, outd, outx = _sc_gather_fn()(ci, ng, nd, nx,
                                             gene_features, pdrug_i32,
                                             cell_i32)
    return _att(outc, outg, outd, outx,
                W_gene, _perm_matrix(), b_gene.reshape(1, D),
                att_w.reshape(2, D))


# speedup vs baseline: 4.3399x; 1.0796x over previous
import functools

import jax
import jax.numpy as jnp
from jax import lax
from jax.experimental import pallas as pl
from jax.experimental.pallas import tpu as pltpu
from jax.experimental.pallas import tpu_sc as plsc

B = 10000
S = 10
D = 128
NG = 100000
ND = 10000
NCELL = 1000
GFD = 128
DFD = 512
DP = D // 2

NCORE = 2
NSUB = 16
NW = NCORE * NSUB
BP = 10240
CH = 32
IDXW = CH * S
ROWS_C0 = 448
ROWS_C1 = 192
NCH0 = ROWS_C0 // CH
NCH1 = ROWS_C1 // CH
BPC = BP + ROWS_C0
MASKHI = -65536


def _sc_body(cidx_hbm, gidx_hbm, didx_hbm, xidx_hbm,
             gene_hbm, pdrug_hbm, cell_hbm,
             outc, outg, outd, outx,
             gidx_v, didx_v, xidx_v, cidx_v,
             bf, b0, b1, accg, accd, accx, cb0, cb1,
             scell,
             semf, semb0, semb1, semc, semog, semod, semox, semoc):
    core = lax.axis_index("c")
    sub = lax.axis_index("s")
    base = jnp.where(core == 0, sub * ROWS_C0,
                     NSUB * ROWS_C0 + sub * ROWS_C1)
    nch = jnp.where(core == 0, NCH0, NCH1)

    pltpu.sync_copy(gidx_hbm.at[pl.ds(base * S, ROWS_C0 * S)], gidx_v)
    pltpu.sync_copy(didx_hbm.at[pl.ds(base * S, ROWS_C0 * S)], didx_v)
    pltpu.sync_copy(xidx_hbm.at[pl.ds(base * S, ROWS_C0 * S)], xidx_v)
    pltpu.sync_copy(cidx_hbm.at[pl.ds(base, ROWS_C0)], cidx_v)

    @pl.when(sub < 8)
    def _():
        pltpu.sync_copy(cell_hbm.at[pl.ds(sub * (NCELL // 8), NCELL // 8)],
                        scell.at[pl.ds(sub * (NCELL // 8), NCELL // 8)])
    plsc.subcore_barrier()

    def issue_item(idx_v, tab, k, buf, sem):
        pltpu.async_copy(tab.at[idx_v.at[pl.ds(k * IDXW, IDXW)]], buf, sem)

    def reduce_f32(buf, acc):
        def red(cc, carry):
            r0 = cc * S
            for dd in range(D // 16):
                v = buf[r0, pl.ds(dd * 16, 16)]
                for s in range(1, S):
                    v = v + buf[r0 + s, pl.ds(dd * 16, 16)]
                acc[cc, pl.ds(dd * 16, 16)] = v
            return carry
        lax.fori_loop(0, CH, red, 0)

    def reduce_packed(buf, acc):
        shift = jnp.full((16,), 65536, jnp.int32)
        mask = jnp.full((16,), MASKHI, jnp.int32)

        def red(cc, carry):
            r0 = cc * S
            for blk in range(DP // 16):
                v = buf[r0, pl.ds(blk * 16, 16)]
                se = lax.bitcast_convert_type(v * shift, jnp.float32)
                so = lax.bitcast_convert_type(v & mask, jnp.float32)
                for s in range(1, S):
                    v = buf[r0 + s, pl.ds(blk * 16, 16)]
                    se = se + lax.bitcast_convert_type(v * shift,
                                                       jnp.float32)
                    so = so + lax.bitcast_convert_type(v & mask,
                                                       jnp.float32)
                acc[cc, pl.ds(blk * 32, 16)] = se
                acc[cc, pl.ds(blk * 32 + 16, 16)] = so
            return carry
        lax.fori_loop(0, CH, red, 0)

    def drain_out(acc, outt, sem):
        pltpu.make_async_copy(acc, outt.at[pl.ds(0, CH)], sem).wait()

    issue_item(gidx_v, gene_hbm, 0, bf, semf)

    def chunk_body(k, carry):
        ob = base + k * CH
        issue_item(didx_v, pdrug_hbm, k, b0, semb0)
        pltpu.make_async_copy(gene_hbm.at[pl.ds(0, IDXW)], bf, semf).wait()

        @pl.when(k > 0)
        def _():
            drain_out(accg, outg, semog)
        reduce_f32(bf, accg)
        pltpu.async_copy(accg, outg.at[pl.ds(ob, CH)], semog)

        issue_item(xidx_v, scell, k, b1, semb1)
        pltpu.make_async_copy(pdrug_hbm.at[pl.ds(0, IDXW)], b0,
                              semb0).wait()

        @pl.when(k > 0)
        def _():
            drain_out(accd, outd, semod)
        reduce_packed(b0, accd)
        pltpu.async_copy(accd, outd.at[pl.ds(ob, CH)], semod)

        @pl.when(k >= 2)
        def _():
            pltpu.make_async_copy(cb0, outc.at[pl.ds(0, CH)], semoc).wait()
        pe = (k % 2) == 0
        po = (k % 2) == 1

        @pl.when(pe)
        def _():
            pltpu.async_copy(gene_hbm.at[cidx_v.at[pl.ds(k * CH, CH)]],
                             cb0, semc)

        @pl.when(po)
        def _():
            pltpu.async_copy(gene_hbm.at[cidx_v.at[pl.ds(k * CH, CH)]],
                             cb1, semc)
        pltpu.make_async_copy(cell_hbm.at[pl.ds(0, IDXW)], b1,
                              semb1).wait()

        @pl.when(k > 0)
        def _():
            drain_out(accx, outx, semox)
        reduce_packed(b1, accx)
        pltpu.async_copy(accx, outx.at[pl.ds(ob, CH)], semox)

        @pl.when(k < nch - 1)
        def _():
            issue_item(gidx_v, gene_hbm, k + 1, bf, semf)
        pltpu.make_async_copy(gene_hbm.at[pl.ds(0, CH)], cb0, semc).wait()

        @pl.when(pe)
        def _():
            pltpu.async_copy(cb0, outc.at[pl.ds(ob, CH)], semoc)

        @pl.when(po)
        def _():
            pltpu.async_copy(cb1, outc.at[pl.ds(ob, CH)], semoc)
        return carry

    lax.fori_loop(0, nch, chunk_body, 0)

    drain_out(accg, outg, semog)
    drain_out(accd, outd, semod)
    drain_out(accx, outx, semox)
    pltpu.make_async_copy(cb0, outc.at[pl.ds(0, CH)], semoc).wait()
    pltpu.make_async_copy(cb1, outc.at[pl.ds(0, CH)], semoc).wait()


@functools.cache
def _sc_gather_fn():
    return pl.kernel(
        _sc_body,
        out_type=(
            jax.ShapeDtypeStruct((BP, D), jnp.float32),
            jax.ShapeDtypeStruct((BP, D), jnp.float32),
            jax.ShapeDtypeStruct((BP, D), jnp.float32),
            jax.ShapeDtypeStruct((BP, D), jnp.float32),
        ),
        mesh=plsc.VectorSubcoreMesh(core_axis_name="c", subcore_axis_name="s",
                                    num_cores=NCORE, num_subcores=NSUB),
        compiler_params=pltpu.CompilerParams(use_tc_tiling_on_sc=False),
        scratch_types=(
            pltpu.VMEM((ROWS_C0 * S,), jnp.int32),
            pltpu.VMEM((ROWS_C0 * S,), jnp.int32),
            pltpu.VMEM((ROWS_C0 * S,), jnp.int32),
            pltpu.VMEM((ROWS_C0,), jnp.int32),
            pltpu.VMEM((IDXW, D), jnp.float32),
            pltpu.VMEM((IDXW, DP), jnp.int32),
            pltpu.VMEM((IDXW, DP), jnp.int32),
            pltpu.VMEM((CH, D), jnp.float32),
            pltpu.VMEM((CH, D), jnp.float32),
            pltpu.VMEM((CH, D), jnp.float32),
            pltpu.VMEM((CH, D), jnp.float32),
            pltpu.VMEM((CH, D), jnp.float32),
            pltpu.VMEM_SHARED((NCELL, DP), jnp.int32),
            pltpu.SemaphoreType.DMA,
            pltpu.SemaphoreType.DMA,
            pltpu.SemaphoreType.DMA,
            pltpu.SemaphoreType.DMA,
            pltpu.SemaphoreType.DMA,
            pltpu.SemaphoreType.DMA,
            pltpu.SemaphoreType.DMA,
            pltpu.SemaphoreType.DMA,
        ),
    )


def _proj_body(x_ref, w_ref, b_ref, o_ref):
    o_ref[...] = (jnp.dot(x_ref[...], w_ref[...],
                          preferred_element_type=jnp.float32)
                  + b_ref[...]).astype(jnp.bfloat16)


_proj_drug = pl.pallas_call(
    _proj_body,
    grid=(5,),
    in_specs=[pl.BlockSpec((ND // 5, DFD), lambda i: (i, 0)),
              pl.BlockSpec((DFD, D), lambda i: (0, 0)),
              pl.BlockSpec((1, D), lambda i: (0, 0))],
    out_specs=pl.BlockSpec((ND // 5, D), lambda i: (i, 0)),
    out_shape=jax.ShapeDtypeStruct((ND, D), jnp.bfloat16),
)


def _att_body(c_ref, g_ref, dr_ref, x_ref, w_ref, p_ref, b_ref, a_ref,
              o_ref):
    inv_s = 1.0 / S
    h = (jnp.dot(c_ref[...], w_ref[...],
                 preferred_element_type=jnp.float32) + b_ref[...])
    ag = (jnp.dot(g_ref[...] * inv_s, w_ref[...],
                  preferred_element_type=jnp.float32) + b_ref[...])
    ad = jnp.dot(dr_ref[...] * inv_s, p_ref[...],
                 preferred_element_type=jnp.float32)
    ax = jnp.dot(x_ref[...] * inv_s, p_ref[...],
                 preferred_element_type=jnp.float32)
    a1 = a_ref[0:1, :]
    a2 = a_ref[1:2, :]

    base = jnp.sum(h * a1, axis=1, keepdims=True)

    def lrelu(v):
        return jnp.where(v >= 0, v, 0.01 * v)

    s0 = lrelu(base + jnp.sum(h * a2, axis=1, keepdims=True))
    s1 = lrelu(base + jnp.sum(ax * a2, axis=1, keepdims=True))
    s2 = lrelu(base + jnp.sum(ad * a2, axis=1, keepdims=True))
    s3 = lrelu(base + jnp.sum(ag * a2, axis=1, keepdims=True))
    m = jnp.maximum(jnp.maximum(s0, s1), jnp.maximum(s2, s3))
    e0 = jnp.exp(s0 - m)
    e1 = jnp.exp(s1 - m)
    e2 = jnp.exp(s2 - m)
    e3 = jnp.exp(s3 - m)
    z = e0 + e1 + e2 + e3
    o_ref[...] = (e0 * h + e1 * ax + e2 * ad + e3 * ag) / z


_att = pl.pallas_call(
    _att_body,
    grid=(10,),
    in_specs=[pl.BlockSpec((B // 10, D), lambda i: (i, 0)),
              pl.BlockSpec((B // 10, D), lambda i: (i, 0)),
              pl.BlockSpec((B // 10, D), lambda i: (i, 0)),
              pl.BlockSpec((B // 10, D), lambda i: (i, 0)),
              pl.BlockSpec((D, D), lambda i: (0, 0)),
              pl.BlockSpec((D, D), lambda i: (0, 0)),
              pl.BlockSpec((1, D), lambda i: (0, 0)),
              pl.BlockSpec((2, D), lambda i: (0, 0))],
    out_specs=pl.BlockSpec((B // 10, D), lambda i: (i, 0)),
    out_shape=jax.ShapeDtypeStruct((B, D), jnp.float32),
)


def _perm_matrix():
    j = jnp.arange(D)
    blk, r = j // 32, j % 32
    src = 32 * blk + jnp.where(r < 16, 2 * r, 2 * (r - 16) + 1)
    return (jnp.arange(D)[None, :] == src[:, None]).astype(jnp.float32)


def kernel(center_ids, neigh_cell, neigh_drug, neigh_gene,
           gene_features, drug_features, cell_embed,
           W_gene, b_gene, W_drug, b_drug, att_w):
    pad = BPC - B
    ci = jnp.pad(center_ids.astype(jnp.int32), (0, pad))
    ng = jnp.pad(neigh_gene.astype(jnp.int32).reshape(-1), (0, pad * S))
    nd = jnp.pad(neigh_drug.astype(jnp.int32).reshape(-1), (0, pad * S))
    nx = jnp.pad(neigh_cell.astype(jnp.int32).reshape(-1), (0, pad * S))

    pdrug = _proj_drug(drug_features, W_drug, b_drug.reshape(1, D))
    pdrug_i32 = lax.bitcast_convert_type(pdrug.reshape(ND, DP, 2),
                                         jnp.int32)
    cell_i32 = lax.bitcast_convert_type(
        cell_embed.astype(jnp.bfloat16).reshape(NCELL, DP, 2), jnp.int32)

    outc, outg, outd, outx = _sc_gather_fn()(ci, ng, nd, nx,
                                             gene_features, pdrug_i32,
                                             cell_i32)
    return _att(outc, outg, outd, outx,
                W_gene, _perm_matrix(), b_gene.reshape(1, D),
                att_w.reshape(2, D))
